# baseline (speedup 1.0000x reference)
import functools

import jax
import jax.numpy as jnp
import numpy as np
from jax import lax
from jax.experimental import pallas as pl
from jax.experimental.pallas import tpu as pltpu
from jax.experimental.pallas import tpu_sc as plsc

N = 10000
E = 160000
C = 4
D = 128
NUM_CLASSES = 25
G = 16

NC = 2
NS = 16
NW = NC * NS
EPW = E // NW
CH = 125
NCHUNK = EPW // CH
HALF = NCHUNK // 2
CHS = 100
NCHS = EPW // CHS
HALFS = NCHS // 2
RPS = N // NS

_BN = 2000
_BE = 2000


def _silu(x):
  return x / (1.0 + jnp.exp(-x))


def _dot(a, b):
  return jnp.dot(a, b, preferred_element_type=jnp.float32)


def _s1_body(t_ref, s_ref, x_ref, emb_ref, wt1, bt1, wt2, bt2, w1a, w1b,
             hn_o, h0_o, t1_o, t2_o):
  t = t_ref[...]
  off = lax.broadcasted_iota(jnp.int32, (_BN, D), 1).astype(jnp.float32) * (
      1.0 / (D - 1))
  d = t - off + 1e-6
  g = jnp.exp((-0.5 * (D - 1) ** 2) * d * d)
  h = jnp.maximum(_dot(g, wt1[...]) + bt1[...], 0.0)
  temb = _dot(h, wt2[...]) + bt2[...]
  lane = lax.broadcasted_iota(jnp.int32, (_BN, 32), 1)
  onehot = (s_ref[...] == lane).astype(jnp.float32)
  h0 = _dot(onehot, emb_ref[...])
  hn = h0 + temb
  hn_o[...] = hn
  h0_o[...] = h0
  x = x_ref[...]
  t1_o[...] = jnp.concatenate([_dot(hn, w1a[...]), x], axis=1)
  t2_o[...] = jnp.concatenate([_dot(hn, w1b[...]), -x], axis=1)


def _node_pre(t2, s2, x16, emb32, wt1, bt1, wt2, bt2, w1a, w1b):
  nb = N // _BN
  row = lambda i: (i, 0)
  rep = lambda i: (0, 0)
  return pl.pallas_call(
      _s1_body,
      grid=(nb,),
      in_specs=[
          pl.BlockSpec((_BN, 1), row),
          pl.BlockSpec((_BN, 1), row),
          pl.BlockSpec((_BN, G), row),
          pl.BlockSpec((32, D), rep),
          pl.BlockSpec((D, D), rep),
          pl.BlockSpec((1, D), rep),
          pl.BlockSpec((D, D), rep),
          pl.BlockSpec((1, D), rep),
          pl.BlockSpec((D, D), rep),
          pl.BlockSpec((D, D), rep),
      ],
      out_specs=[
          pl.BlockSpec((_BN, D), row),
          pl.BlockSpec((_BN, D), row),
          pl.BlockSpec((_BN, D + G), row),
          pl.BlockSpec((_BN, D + G), row),
      ],
      out_shape=[
          jax.ShapeDtypeStruct((N, D), jnp.float32),
          jax.ShapeDtypeStruct((N, D), jnp.float32),
          jax.ShapeDtypeStruct((N, D + G), jnp.float32),
          jax.ShapeDtypeStruct((N, D + G), jnp.float32),
      ],
  )(t2, s2, x16, emb32, wt1, bt1, wt2, bt2, w1a, w1b)


@functools.cache
def _sc_mesh():
  return plsc.VectorSubcoreMesh(
      core_axis_name="c", subcore_axis_name="s", num_cores=NC, num_subcores=NS)


@functools.cache
def _make_sc_gather():
  @functools.partial(
      pl.kernel,
      out_type=[
          jax.ShapeDtypeStruct((E, D), jnp.float32),
          jax.ShapeDtypeStruct((E, G), jnp.float32),
      ],
      mesh=_sc_mesh(),
      compiler_params=pltpu.CompilerParams(use_tc_tiling_on_sc=False),
      scratch_types=[
          pltpu.VMEM((NCHUNK, CH), jnp.int32),
          pltpu.VMEM((NCHUNK, CH), jnp.int32),
          pltpu.VMEM((CH, D + G), jnp.float32),
          pltpu.VMEM((CH, D + G), jnp.float32),
          pltpu.VMEM((CH, D + G), jnp.float32),
          pltpu.VMEM((CH, D + G), jnp.float32),
          pltpu.VMEM((CH, D), jnp.float32),
          pltpu.VMEM((CH, G), jnp.float32),
          pltpu.SemaphoreType.DMA,
          pltpu.SemaphoreType.DMA,
      ],
  )
  def _sc_gather_k(t1_hbm, t2_hbm, src3_hbm, dst3_hbm,
                   ms_out, cd_out,
                   si_all, di_all, a0, b0, a1, b1, ms_v, cd_v,
                   sem0, sem1):
    wid = lax.axis_index("s") * NC + lax.axis_index("c")
    base = wid * EPW
    pltpu.sync_copy(src3_hbm.at[wid], si_all)
    pltpu.sync_copy(dst3_hbm.at[wid], di_all)

    def start(ch, a_v, b_v, sem):
      pltpu.async_copy(t1_hbm.at[si_all.at[ch]], a_v, sem)
      pltpu.async_copy(t2_hbm.at[di_all.at[ch]], b_v, sem)

    def wait(a_v, b_v, sem):
      pltpu.make_async_copy(t1_hbm.at[si_all.at[0]], a_v, sem).wait()
      pltpu.make_async_copy(t2_hbm.at[di_all.at[0]], b_v, sem).wait()

    def add_store(ch, a_v, b_v):
      def add_row(r5, carry):
        for rr in range(5):
          r = r5 * 5 + rr
          for k in range(D // 16):
            sl = pl.ds(k * 16, 16)
            ms_v[r, sl] = a_v[r, sl] + b_v[r, sl]
          cd_v[r, :] = a_v[r, pl.ds(D, G)] + b_v[r, pl.ds(D, G)]
        return carry

      lax.fori_loop(0, CH // 5, add_row, None)
      pltpu.sync_copy(ms_v, ms_out.at[pl.ds(base + ch * CH, CH)])
      pltpu.sync_copy(cd_v, cd_out.at[pl.ds(base + ch * CH, CH)])

    start(0, a0, b0, sem0)

    def pair(j, _):
      c0 = 2 * j
      start(c0 + 1, a1, b1, sem1)
      wait(a0, b0, sem0)
      add_store(c0, a0, b0)

      @pl.when(j < HALF - 1)
      def _prefetch():
        start(c0 + 2, a0, b0, sem0)

      wait(a1, b1, sem1)
      add_store(c0 + 1, a1, b1)
      return _

    lax.fori_loop(0, HALF, pair, None)

  return _sc_gather_k


def _sc_gather(t1, t2_tab, src3, dst3):
  return _make_sc_gather()(t1, t2_tab, src3, dst3)


def _s3_body(ms_ref, cd_ref, a16, b16, wr48, br, w1c, be1, we2, be2, wc,
             m_o, cm_o):
  z = cd_ref[...]
  prod = _dot(z, a16[...]) * _dot(z, b16[...])
  rfeat = _silu(_dot(prod, wr48[...]) + br[...])
  u = ms_ref[...] + _dot(rfeat, w1c[...]) + be1[...]
  m = _silu(_dot(_silu(u), we2[...]) + be2[...])
  wgt = jnp.sum(m * wc[...], axis=1, keepdims=True)
  cm = z * wgt
  lane = lax.broadcasted_iota(jnp.int32, (_BE, G), 1)
  m_o[...] = m
  cm_o[...] = jnp.where(lane == 12, 1.0, cm)


def _edge_mlp(ms, cd, a16, b16, wr48, br, w1c, be1, we2, be2, wc):
  nb = E // _BE
  row = lambda i: (i, 0)
  rep = lambda i: (0, 0)
  return pl.pallas_call(
      _s3_body,
      grid=(nb,),
      in_specs=[
          pl.BlockSpec((_BE, D), row),
          pl.BlockSpec((_BE, G), row),
          pl.BlockSpec((G, D), rep),
          pl.BlockSpec((G, D), rep),
          pl.BlockSpec((D, D), rep),
          pl.BlockSpec((1, D), rep),
          pl.BlockSpec((D, D), rep),
          pl.BlockSpec((1, D), rep),
          pl.BlockSpec((D, D), rep),
          pl.BlockSpec((1, D), rep),
          pl.BlockSpec((1, D), rep),
      ],
      out_specs=[
          pl.BlockSpec((_BE, D), row),
          pl.BlockSpec((_BE, G), row),
      ],
      out_shape=[
          jax.ShapeDtypeStruct((E, D), jnp.float32),
          jax.ShapeDtypeStruct((E, G), jnp.float32),
      ],
  )(ms, cd, a16, b16, wr48, br, w1c, be1, we2, be2, wc)


@functools.cache
def _make_sc_scatter():
  @functools.partial(
      pl.kernel,
      out_type=[
          jax.ShapeDtypeStruct((NC, N, D), jnp.float32),
          jax.ShapeDtypeStruct((NC, N, G), jnp.float32),
      ],
      mesh=_sc_mesh(),
      compiler_params=pltpu.CompilerParams(use_tc_tiling_on_sc=False),
      scratch_types=[
          pltpu.VMEM((NCHS, CHS), jnp.int32),
          pltpu.VMEM((CHS, D), jnp.float32),
          pltpu.VMEM((CHS, D), jnp.float32),
          pltpu.VMEM((CHS, G), jnp.float32),
          pltpu.VMEM((CHS, G), jnp.float32),
          pltpu.VMEM_SHARED((N, D), jnp.float32),
          pltpu.VMEM_SHARED((N, G), jnp.float32),
          pltpu.SemaphoreType.DMA,
          pltpu.SemaphoreType.DMA,
      ],
  )
  def _sc_scatter_k(m_hbm, cm_hbm, dst3_hbm, zd_hbm, zg_hbm,
                    outd_hbm, outg_hbm,
                    idx_all, m0, m1, c0_v, c1_v, shd, shg, sem0, sem1):
    c = lax.axis_index("c")
    s = lax.axis_index("s")
    wid = s * NC + c
    rows0 = s * RPS
    pltpu.sync_copy(zd_hbm.at[pl.ds(rows0, RPS)], shd.at[pl.ds(rows0, RPS)])
    pltpu.sync_copy(zg_hbm.at[pl.ds(rows0, RPS)], shg.at[pl.ds(rows0, RPS)])
    pltpu.sync_copy(dst3_hbm.at[wid], idx_all)
    plsc.subcore_barrier()

    base = wid * EPW

    def start(ch, m_v, c_v, sem):
      pltpu.async_copy(m_hbm.at[pl.ds(base + ch * CHS, CHS)], m_v, sem)
      pltpu.async_copy(cm_hbm.at[pl.ds(base + ch * CHS, CHS)], c_v, sem)

    def wait(m_v, c_v, sem):
      pltpu.make_async_copy(m_hbm.at[pl.ds(0, CHS)], m_v, sem).wait()
      pltpu.make_async_copy(cm_hbm.at[pl.ds(0, CHS)], c_v, sem).wait()

    start(0, m0, c0_v, sem0)

    def pair(j, _):
      c0 = 2 * j
      start(c0 + 1, m1, c1_v, sem1)
      wait(m0, c0_v, sem0)
      pltpu.sync_copy(m0, shd.at[idx_all.at[c0]], add=True)
      pltpu.sync_copy(c0_v, shg.at[idx_all.at[c0]], add=True)

      @pl.when(j < HALFS - 1)
      def _prefetch():
        start(c0 + 2, m0, c0_v, sem0)

      wait(m1, c1_v, sem1)
      pltpu.sync_copy(m1, shd.at[idx_all.at[c0 + 1]], add=True)
      pltpu.sync_copy(c1_v, shg.at[idx_all.at[c0 + 1]], add=True)
      return _

    lax.fori_loop(0, HALFS, pair, None)
    plsc.subcore_barrier()
    pltpu.sync_copy(shd.at[pl.ds(rows0, RPS)], outd_hbm.at[c, pl.ds(rows0, RPS)])
    pltpu.sync_copy(shg.at[pl.ds(rows0, RPS)], outg_hbm.at[c, pl.ds(rows0, RPS)])

  return _sc_scatter_k


def _sc_scatter(m, cm, dst3s, zd, zg):
  return _make_sc_scatter()(m, cm, dst3s, zd, zg)


def _s5_body(hn_ref, h0_ref, x_ref, pd_ref, pg_ref, wn1a, wn1b, bn1, wn2, bn2,
             wp, bp, wf1, bf1, wf2, logits_o, xout_o):
  hn = hn_ref[...]
  h0 = h0_ref[...]
  pd = pd_ref[...]
  pg = pg_ref[...]
  agg = pd[0] + pd[1]
  cm = pg[0] + pg[1]
  hmid = _silu(_dot(hn, wn1a[...]) + _dot(agg, wn1b[...]) + bn1[...])
  hout = hn + _dot(hmid, wn2[...]) + bn2[...]
  cnt = jnp.maximum(cm[:, 12:13], 1.0)
  lane = lax.broadcasted_iota(jnp.int32, (_BN, G), 1)
  xout_o[...] = x_ref[...] + jnp.where(lane < 12, cm / cnt, 0.0)
  proj = _dot(_silu(hout), wp[...]) + bp[...]
  gate = h0 * jax.nn.sigmoid(proj)
  l1 = _silu(_dot(_silu(gate), wf1[...]) + bf1[...])
  logits_o[...] = _dot(l1, wf2[...])


def _node_post(hn, h0, x16, pd, pg, wn1a, wn1b, bn1, wn2, bn2,
               wp, bp, wf1, bf1, wf2):
  nb = N // _BN
  row = lambda i: (i, 0)
  rep = lambda i: (0, 0)
  return pl.pallas_call(
      _s5_body,
      grid=(nb,),
      in_specs=[
          pl.BlockSpec((_BN, D), row),
          pl.BlockSpec((_BN, D), row),
          pl.BlockSpec((_BN, G), row),
          pl.BlockSpec((NC, _BN, D), lambda i: (0, i, 0)),
          pl.BlockSpec((NC, _BN, G), lambda i: (0, i, 0)),
          pl.BlockSpec((D, D), rep),
          pl.BlockSpec((D, D), rep),
          pl.BlockSpec((1, D), rep),
          pl.BlockSpec((D, D), rep),
          pl.BlockSpec((1, D), rep),
          pl.BlockSpec((D, D), rep),
          pl.BlockSpec((1, D), rep),
          pl.BlockSpec((D, D), rep),
          pl.BlockSpec((1, D), rep),
          pl.BlockSpec((D, D), rep),
      ],
      out_specs=[
          pl.BlockSpec((_BN, D), row),
          pl.BlockSpec((_BN, G), row),
      ],
      out_shape=[
          jax.ShapeDtypeStruct((N, D), jnp.float32),
          jax.ShapeDtypeStruct((N, G), jnp.float32),
      ],
  )(hn, h0, x16, pd, pg, wn1a, wn1b, bn1, wn2, bn2, wp, bp, wf1, bf1, wf2)


def _radial_selectors():
  a = np.zeros((G, D), np.float32)
  b = np.zeros((G, D), np.float32)
  for c in range(4):
    for dd in range(4):
      for i in range(3):
        p = (c * 4 + dd) * 3 + i
        a[3 * c + i, p] = 1.0
        b[3 * dd + i, p] = 1.0
  return a, b


_A16_NP, _B16_NP = _radial_selectors()
_CD_IDX = np.repeat(np.arange(16), 3)


def kernel(X, t, S, edge_index, params):
  p = params
  x16 = jnp.pad(X.reshape(N, 12), ((0, 0), (0, 4)))
  t2 = t.reshape(N, 1).astype(jnp.float32)
  s2 = S.reshape(N, 1).astype(jnp.int32)
  src = edge_index[0].astype(jnp.int32)
  dst = edge_index[1].astype(jnp.int32)
  src3 = src.reshape(NW, NCHUNK, CH)
  dst3 = dst.reshape(NW, NCHUNK, CH)
  dst3s = dst.reshape(NW, NCHS, CHS)

  emb32 = jnp.pad(p["embed"], ((0, 32 - (NUM_CLASSES + 1)), (0, 0)))
  w1a = p["W_e1"][:D]
  w1b = p["W_e1"][D:2 * D]
  w1c = p["W_e1"][2 * D:]
  wr48 = jnp.pad(p["W_r"][_CD_IDX], ((0, D - 48), (0, 0)))
  row128 = lambda v: v.reshape(1, D)

  hn, h0, t1, t2_tab = _node_pre(
      t2, s2, x16, emb32,
      p["W_t1"], row128(p["b_t1"]), p["W_t2"], row128(p["b_t2"]), w1a, w1b)

  ms, cd = _sc_gather(t1, t2_tab, src3, dst3)

  m, cm = _edge_mlp(
      ms, cd, jnp.asarray(_A16_NP), jnp.asarray(_B16_NP), wr48,
      row128(p["b_r"]), w1c, row128(p["b_e1"]),
      p["W_e2"], row128(p["b_e2"]), p["W_c"].reshape(1, D))

  pd, pg = _sc_scatter(m, cm, dst3s,
                       jnp.zeros((N, D), jnp.float32),
                       jnp.zeros((N, G), jnp.float32))

  wf2 = jnp.pad(p["W_f2"], ((0, 0), (0, D - NUM_CLASSES)))
  logits128, xout16 = _node_post(
      hn, h0, x16, pd, pg,
      p["W_n1"][:D], p["W_n1"][D:], row128(p["b_n1"]),
      p["W_n2"], row128(p["b_n2"]),
      p["W_p"], row128(p["b_p"]),
      p["W_f1"], row128(p["b_f1"]), wf2)

  logits = logits128[:, :NUM_CLASSES] + p["b_f2"]
  xout = xout16[:, :12].reshape(N, C, 3)
  return logits, xout


# speedup vs baseline: 1.0429x; 1.0429x over previous
import functools

import jax
import jax.numpy as jnp
import numpy as np
from jax import lax
from jax.experimental import pallas as pl
from jax.experimental.pallas import tpu as pltpu
from jax.experimental.pallas import tpu_sc as plsc

N = 10000
E = 160000
C = 4
D = 128
NUM_CLASSES = 25
G = 16

NC = 2
NS = 16
NW = NC * NS
EPW = E // NW
CH = 125
NCHUNK = EPW // CH
HALF = NCHUNK // 2
CHS = 100
NCHS = EPW // CHS
HALFS = NCHS // 2
RPS = N // NS

_BN = 2000
_BE = 2000


def _silu(x):
  return x / (1.0 + jnp.exp(-x))


def _dot(a, b):
  return jnp.dot(a, b, preferred_element_type=jnp.float32)


def _s1_body(t_ref, s_ref, x_ref, emb_ref, wt1, bt1, wt2, bt2, w1a, w1b,
             hn_o, h0_o, t1_o, t2_o):
  t = t_ref[...]
  off = lax.broadcasted_iota(jnp.int32, (_BN, D), 1).astype(jnp.float32) * (
      1.0 / (D - 1))
  d = t - off + 1e-6
  g = jnp.exp((-0.5 * (D - 1) ** 2) * d * d)
  h = jnp.maximum(_dot(g, wt1[...]) + bt1[...], 0.0)
  temb = _dot(h, wt2[...]) + bt2[...]
  lane = lax.broadcasted_iota(jnp.int32, (_BN, 32), 1)
  onehot = (s_ref[...] == lane).astype(jnp.float32)
  h0 = _dot(onehot, emb_ref[...])
  hn = h0 + temb
  hn_o[...] = hn
  h0_o[...] = h0
  x = x_ref[...]
  t1_o[...] = jnp.concatenate([_dot(hn, w1a[...]), x], axis=1)
  t2_o[...] = jnp.concatenate([_dot(hn, w1b[...]), -x], axis=1)


def _node_pre(t2, s2, x16, emb32, wt1, bt1, wt2, bt2, w1a, w1b):
  nb = N // _BN
  row = lambda i: (i, 0)
  rep = lambda i: (0, 0)
  return pl.pallas_call(
      _s1_body,
      grid=(nb,),
      in_specs=[
          pl.BlockSpec((_BN, 1), row),
          pl.BlockSpec((_BN, 1), row),
          pl.BlockSpec((_BN, G), row),
          pl.BlockSpec((32, D), rep),
          pl.BlockSpec((D, D), rep),
          pl.BlockSpec((1, D), rep),
          pl.BlockSpec((D, D), rep),
          pl.BlockSpec((1, D), rep),
          pl.BlockSpec((D, D), rep),
          pl.BlockSpec((D, D), rep),
      ],
      out_specs=[
          pl.BlockSpec((_BN, D), row),
          pl.BlockSpec((_BN, D), row),
          pl.BlockSpec((_BN, D + G), row),
          pl.BlockSpec((_BN, D + G), row),
      ],
      out_shape=[
          jax.ShapeDtypeStruct((N, D), jnp.float32),
          jax.ShapeDtypeStruct((N, D), jnp.float32),
          jax.ShapeDtypeStruct((N, D + G), jnp.float32),
          jax.ShapeDtypeStruct((N, D + G), jnp.float32),
      ],
  )(t2, s2, x16, emb32, wt1, bt1, wt2, bt2, w1a, w1b)


@functools.cache
def _sc_mesh():
  return plsc.VectorSubcoreMesh(
      core_axis_name="c", subcore_axis_name="s", num_cores=NC, num_subcores=NS)


@functools.cache
def _make_sc_gather():
  @functools.partial(
      pl.kernel,
      out_type=[
          jax.ShapeDtypeStruct((E, D), jnp.float32),
          jax.ShapeDtypeStruct((E, G), jnp.float32),
      ],
      mesh=_sc_mesh(),
      compiler_params=pltpu.CompilerParams(use_tc_tiling_on_sc=False),
      scratch_types=[
          pltpu.VMEM((NCHUNK, CH), jnp.int32),
          pltpu.VMEM((NCHUNK, CH), jnp.int32),
          pltpu.VMEM((CH, D + G), jnp.float32),
          pltpu.VMEM((CH, D + G), jnp.float32),
          pltpu.VMEM((CH, D + G), jnp.float32),
          pltpu.VMEM((CH, D + G), jnp.float32),
          pltpu.VMEM((CH, D), jnp.float32),
          pltpu.VMEM((CH, G), jnp.float32),
          pltpu.VMEM((CH, D), jnp.float32),
          pltpu.VMEM((CH, G), jnp.float32),
          pltpu.SemaphoreType.DMA,
          pltpu.SemaphoreType.DMA,
          pltpu.SemaphoreType.DMA,
          pltpu.SemaphoreType.DMA,
      ],
  )
  def _sc_gather_k(t1_hbm, t2_hbm, src3_hbm, dst3_hbm,
                   ms_out, cd_out,
                   si_all, di_all, a0, b0, a1, b1, ms0, cd0, ms1, cd1,
                   sem0, sem1, semS0, semS1):
    wid = lax.axis_index("s") * NC + lax.axis_index("c")
    base = wid * EPW
    pltpu.sync_copy(src3_hbm.at[wid], si_all)
    pltpu.sync_copy(dst3_hbm.at[wid], di_all)

    def start(ch, a_v, b_v, sem):
      pltpu.async_copy(t1_hbm.at[si_all.at[ch]], a_v, sem)
      pltpu.async_copy(t2_hbm.at[di_all.at[ch]], b_v, sem)

    def wait(a_v, b_v, sem):
      pltpu.make_async_copy(t1_hbm.at[si_all.at[0]], a_v, sem).wait()
      pltpu.make_async_copy(t2_hbm.at[di_all.at[0]], b_v, sem).wait()

    def add(a_v, b_v, ms_v, cd_v):
      def add_row(r5, carry):
        for rr in range(5):
          r = r5 * 5 + rr
          for k in range(D // 16):
            sl = pl.ds(k * 16, 16)
            ms_v[r, sl] = a_v[r, sl] + b_v[r, sl]
          cd_v[r, :] = a_v[r, pl.ds(D, G)] + b_v[r, pl.ds(D, G)]
        return carry

      lax.fori_loop(0, CH // 5, add_row, None)

    def astore(ch, ms_v, cd_v, sem):
      pltpu.async_copy(ms_v, ms_out.at[pl.ds(base + ch * CH, CH)], sem)
      pltpu.async_copy(cd_v, cd_out.at[pl.ds(base + ch * CH, CH)], sem)

    def wait_store(ms_v, cd_v, sem):
      pltpu.make_async_copy(ms_out.at[pl.ds(0, CH)], ms_v, sem).wait()
      pltpu.make_async_copy(cd_out.at[pl.ds(0, CH)], cd_v, sem).wait()

    start(0, a0, b0, sem0)
    start(1, a1, b1, sem1)
    wait(a0, b0, sem0)
    add(a0, b0, ms0, cd0)
    astore(0, ms0, cd0, semS0)
    start(2, a0, b0, sem0)
    wait(a1, b1, sem1)
    add(a1, b1, ms1, cd1)
    astore(1, ms1, cd1, semS1)

    def pair(j, _):
      c0 = 2 * j
      start(c0 + 1, a1, b1, sem1)
      wait(a0, b0, sem0)
      wait_store(ms0, cd0, semS0)
      add(a0, b0, ms0, cd0)
      astore(c0, ms0, cd0, semS0)

      @pl.when(j < HALF - 1)
      def _prefetch():
        start(c0 + 2, a0, b0, sem0)

      wait(a1, b1, sem1)
      wait_store(ms1, cd1, semS1)
      add(a1, b1, ms1, cd1)
      astore(c0 + 1, ms1, cd1, semS1)
      return _

    lax.fori_loop(1, HALF, pair, None)
    wait_store(ms0, cd0, semS0)
    wait_store(ms1, cd1, semS1)

  return _sc_gather_k


def _sc_gather(t1, t2_tab, src3, dst3):
  return _make_sc_gather()(t1, t2_tab, src3, dst3)


def _s3_body(ms_ref, cd_ref, a16, b16, wr48, br, w1c, be1, we2, be2, wc,
             m_o, cm_o):
  z = cd_ref[...]
  prod = _dot(z, a16[...]) * _dot(z, b16[...])
  rfeat = _silu(_dot(prod, wr48[...]) + br[...])
  u = ms_ref[...] + _dot(rfeat, w1c[...]) + be1[...]
  m = _silu(_dot(_silu(u), we2[...]) + be2[...])
  wgt = jnp.sum(m * wc[...], axis=1, keepdims=True)
  cm = z * wgt
  lane = lax.broadcasted_iota(jnp.int32, (_BE, G), 1)
  m_o[...] = m
  cm_o[...] = jnp.where(lane == 12, 1.0, cm)


def _edge_mlp(ms, cd, a16, b16, wr48, br, w1c, be1, we2, be2, wc):
  nb = E // _BE
  row = lambda i: (i, 0)
  rep = lambda i: (0, 0)
  return pl.pallas_call(
      _s3_body,
      grid=(nb,),
      in_specs=[
          pl.BlockSpec((_BE, D), row),
          pl.BlockSpec((_BE, G), row),
          pl.BlockSpec((G, D), rep),
          pl.BlockSpec((G, D), rep),
          pl.BlockSpec((D, D), rep),
          pl.BlockSpec((1, D), rep),
          pl.BlockSpec((D, D), rep),
          pl.BlockSpec((1, D), rep),
          pl.BlockSpec((D, D), rep),
          pl.BlockSpec((1, D), rep),
          pl.BlockSpec((1, D), rep),
      ],
      out_specs=[
          pl.BlockSpec((_BE, D), row),
          pl.BlockSpec((_BE, G), row),
      ],
      out_shape=[
          jax.ShapeDtypeStruct((E, D), jnp.float32),
          jax.ShapeDtypeStruct((E, G), jnp.float32),
      ],
  )(ms, cd, a16, b16, wr48, br, w1c, be1, we2, be2, wc)


@functools.cache
def _make_sc_scatter():
  @functools.partial(
      pl.kernel,
      out_type=[
          jax.ShapeDtypeStruct((NC, N, D), jnp.float32),
          jax.ShapeDtypeStruct((NC, N, G), jnp.float32),
      ],
      mesh=_sc_mesh(),
      compiler_params=pltpu.CompilerParams(use_tc_tiling_on_sc=False),
      scratch_types=[
          pltpu.VMEM((NCHS, CHS), jnp.int32),
          pltpu.VMEM((CHS, D), jnp.float32),
          pltpu.VMEM((CHS, D), jnp.float32),
          pltpu.VMEM((CHS, G), jnp.float32),
          pltpu.VMEM((CHS, G), jnp.float32),
          pltpu.VMEM_SHARED((N, D), jnp.float32),
          pltpu.VMEM_SHARED((N, G), jnp.float32),
          pltpu.SemaphoreType.DMA,
          pltpu.SemaphoreType.DMA,
      ],
  )
  def _sc_scatter_k(m_hbm, cm_hbm, dst3_hbm, zd_hbm, zg_hbm,
                    outd_hbm, outg_hbm,
                    idx_all, m0, m1, c0_v, c1_v, shd, shg, sem0, sem1):
    c = lax.axis_index("c")
    s = lax.axis_index("s")
    wid = s * NC + c
    rows0 = s * RPS
    pltpu.sync_copy(zd_hbm.at[pl.ds(rows0, RPS)], shd.at[pl.ds(rows0, RPS)])
    pltpu.sync_copy(zg_hbm.at[pl.ds(rows0, RPS)], shg.at[pl.ds(rows0, RPS)])
    pltpu.sync_copy(dst3_hbm.at[wid], idx_all)
    plsc.subcore_barrier()

    base = wid * EPW

    def start(ch, m_v, c_v, sem):
      pltpu.async_copy(m_hbm.at[pl.ds(base + ch * CHS, CHS)], m_v, sem)
      pltpu.async_copy(cm_hbm.at[pl.ds(base + ch * CHS, CHS)], c_v, sem)

    def wait(m_v, c_v, sem):
      pltpu.make_async_copy(m_hbm.at[pl.ds(0, CHS)], m_v, sem).wait()
      pltpu.make_async_copy(cm_hbm.at[pl.ds(0, CHS)], c_v, sem).wait()

    start(0, m0, c0_v, sem0)

    def pair(j, _):
      c0 = 2 * j
      start(c0 + 1, m1, c1_v, sem1)
      wait(m0, c0_v, sem0)
      pltpu.sync_copy(m0, shd.at[idx_all.at[c0]], add=True)
      pltpu.sync_copy(c0_v, shg.at[idx_all.at[c0]], add=True)

      @pl.when(j < HALFS - 1)
      def _prefetch():
        start(c0 + 2, m0, c0_v, sem0)

      wait(m1, c1_v, sem1)
      pltpu.sync_copy(m1, shd.at[idx_all.at[c0 + 1]], add=True)
      pltpu.sync_copy(c1_v, shg.at[idx_all.at[c0 + 1]], add=True)
      return _

    lax.fori_loop(0, HALFS, pair, None)
    plsc.subcore_barrier()
    pltpu.sync_copy(shd.at[pl.ds(rows0, RPS)], outd_hbm.at[c, pl.ds(rows0, RPS)])
    pltpu.sync_copy(shg.at[pl.ds(rows0, RPS)], outg_hbm.at[c, pl.ds(rows0, RPS)])

  return _sc_scatter_k


def _sc_scatter(m, cm, dst3s, zd, zg):
  return _make_sc_scatter()(m, cm, dst3s, zd, zg)


def _s5_body(hn_ref, h0_ref, x_ref, pd_ref, pg_ref, wn1a, wn1b, bn1, wn2, bn2,
             wp, bp, wf1, bf1, wf2, logits_o, xout_o):
  hn = hn_ref[...]
  h0 = h0_ref[...]
  pd = pd_ref[...]
  pg = pg_ref[...]
  agg = pd[0] + pd[1]
  cm = pg[0] + pg[1]
  hmid = _silu(_dot(hn, wn1a[...]) + _dot(agg, wn1b[...]) + bn1[...])
  hout = hn + _dot(hmid, wn2[...]) + bn2[...]
  cnt = jnp.maximum(cm[:, 12:13], 1.0)
  lane = lax.broadcasted_iota(jnp.int32, (_BN, G), 1)
  xout_o[...] = x_ref[...] + jnp.where(lane < 12, cm / cnt, 0.0)
  proj = _dot(_silu(hout), wp[...]) + bp[...]
  gate = h0 * jax.nn.sigmoid(proj)
  l1 = _silu(_dot(_silu(gate), wf1[...]) + bf1[...])
  logits_o[...] = _dot(l1, wf2[...])


def _node_post(hn, h0, x16, pd, pg, wn1a, wn1b, bn1, wn2, bn2,
               wp, bp, wf1, bf1, wf2):
  nb = N // _BN
  row = lambda i: (i, 0)
  rep = lambda i: (0, 0)
  return pl.pallas_call(
      _s5_body,
      grid=(nb,),
      in_specs=[
          pl.BlockSpec((_BN, D), row),
          pl.BlockSpec((_BN, D), row),
          pl.BlockSpec((_BN, G), row),
          pl.BlockSpec((NC, _BN, D), lambda i: (0, i, 0)),
          pl.BlockSpec((NC, _BN, G), lambda i: (0, i, 0)),
          pl.BlockSpec((D, D), rep),
          pl.BlockSpec((D, D), rep),
          pl.BlockSpec((1, D), rep),
          pl.BlockSpec((D, D), rep),
          pl.BlockSpec((1, D), rep),
          pl.BlockSpec((D, D), rep),
          pl.BlockSpec((1, D), rep),
          pl.BlockSpec((D, D), rep),
          pl.BlockSpec((1, D), rep),
          pl.BlockSpec((D, D), rep),
      ],
      out_specs=[
          pl.BlockSpec((_BN, D), row),
          pl.BlockSpec((_BN, G), row),
      ],
      out_shape=[
          jax.ShapeDtypeStruct((N, D), jnp.float32),
          jax.ShapeDtypeStruct((N, G), jnp.float32),
      ],
  )(hn, h0, x16, pd, pg, wn1a, wn1b, bn1, wn2, bn2, wp, bp, wf1, bf1, wf2)


def _radial_selectors():
  a = np.zeros((G, D), np.float32)
  b = np.zeros((G, D), np.float32)
  for c in range(4):
    for dd in range(4):
      for i in range(3):
        p = (c * 4 + dd) * 3 + i
        a[3 * c + i, p] = 1.0
        b[3 * dd + i, p] = 1.0
  return a, b


_A16_NP, _B16_NP = _radial_selectors()
_CD_IDX = np.repeat(np.arange(16), 3)


def kernel(X, t, S, edge_index, params):
  p = params
  x16 = jnp.pad(X.reshape(N, 12), ((0, 0), (0, 4)))
  t2 = t.reshape(N, 1).astype(jnp.float32)
  s2 = S.reshape(N, 1).astype(jnp.int32)
  src = edge_index[0].astype(jnp.int32)
  dst = edge_index[1].astype(jnp.int32)
  src3 = src.reshape(NW, NCHUNK, CH)
  dst3 = dst.reshape(NW, NCHUNK, CH)
  dst3s = dst.reshape(NW, NCHS, CHS)

  emb32 = jnp.pad(p["embed"], ((0, 32 - (NUM_CLASSES + 1)), (0, 0)))
  w1a = p["W_e1"][:D]
  w1b = p["W_e1"][D:2 * D]
  w1c = p["W_e1"][2 * D:]
  wr48 = jnp.pad(p["W_r"][_CD_IDX], ((0, D - 48), (0, 0)))
  row128 = lambda v: v.reshape(1, D)

  hn, h0, t1, t2_tab = _node_pre(
      t2, s2, x16, emb32,
      p["W_t1"], row128(p["b_t1"]), p["W_t2"], row128(p["b_t2"]), w1a, w1b)

  ms, cd = _sc_gather(t1, t2_tab, src3, dst3)

  m, cm = _edge_mlp(
      ms, cd, jnp.asarray(_A16_NP), jnp.asarray(_B16_NP), wr48,
      row128(p["b_r"]), w1c, row128(p["b_e1"]),
      p["W_e2"], row128(p["b_e2"]), p["W_c"].reshape(1, D))

  pd, pg = _sc_scatter(m, cm, dst3s,
                       jnp.zeros((N, D), jnp.float32),
                       jnp.zeros((N, G), jnp.float32))

  wf2 = jnp.pad(p["W_f2"], ((0, 0), (0, D - NUM_CLASSES)))
  logits128, xout16 = _node_post(
      hn, h0, x16, pd, pg,
      p["W_n1"][:D], p["W_n1"][D:], row128(p["b_n1"]),
      p["W_n2"], row128(p["b_n2"]),
      p["W_p"], row128(p["b_p"]),
      p["W_f1"], row128(p["b_f1"]), wf2)

  logits = logits128[:, :NUM_CLASSES] + p["b_f2"]
  xout = xout16[:, :12].reshape(N, C, 3)
  return logits, xout


# speedup vs baseline: 1.3350x; 1.2801x over previous
import functools

import jax
import jax.numpy as jnp
import numpy as np
from jax import lax
from jax.experimental import pallas as pl
from jax.experimental.pallas import tpu as pltpu
from jax.experimental.pallas import tpu_sc as plsc

N = 10000
E = 160000
C = 4
D = 128
NUM_CLASSES = 25
G = 16

NC = 2
NS = 16
NW = NC * NS
EPW = E // NW
CH = 125
NCHUNK = EPW // CH
HALF = NCHUNK // 2
CHS = 100
NCHS = EPW // CHS
HALFS = NCHS // 2
RPS = N // NS

_BN = 2000
_BE = 4000


def _silu(x):
  return x / (1.0 + jnp.exp(-x))


def _dot(a, b):
  return jnp.dot(a, b, preferred_element_type=jnp.float32)


def _s1_body(t_ref, s_ref, x_ref, emb_ref, wt1, bt1, wt2, bt2, w1a, w1b,
             hn_o, h0_o, t1_o, t2_o):
  t = t_ref[...]
  off = lax.broadcasted_iota(jnp.int32, (_BN, D), 1).astype(jnp.float32) * (
      1.0 / (D - 1))
  d = t - off + 1e-6
  g = jnp.exp((-0.5 * (D - 1) ** 2) * d * d)
  h = jnp.maximum(_dot(g, wt1[...]) + bt1[...], 0.0)
  temb = _dot(h, wt2[...]) + bt2[...]
  lane = lax.broadcasted_iota(jnp.int32, (_BN, 32), 1)
  onehot = (s_ref[...] == lane).astype(jnp.float32)
  h0 = _dot(onehot, emb_ref[...])
  hn = h0 + temb
  hn_o[...] = hn
  h0_o[...] = h0
  x = x_ref[...]
  t1_o[...] = jnp.concatenate([_dot(hn, w1a[...]), x], axis=1)
  t2_o[...] = jnp.concatenate([_dot(hn, w1b[...]), -x], axis=1)


def _node_pre(t2, s2, x16, emb32, wt1, bt1, wt2, bt2, w1a, w1b):
  nb = N // _BN
  row = lambda i: (i, 0)
  rep = lambda i: (0, 0)
  return pl.pallas_call(
      _s1_body,
      grid=(nb,),
      in_specs=[
          pl.BlockSpec((_BN, 1), row),
          pl.BlockSpec((_BN, 1), row),
          pl.BlockSpec((_BN, G), row),
          pl.BlockSpec((32, D), rep),
          pl.BlockSpec((D, D), rep),
          pl.BlockSpec((1, D), rep),
          pl.BlockSpec((D, D), rep),
          pl.BlockSpec((1, D), rep),
          pl.BlockSpec((D, D), rep),
          pl.BlockSpec((D, D), rep),
      ],
      out_specs=[
          pl.BlockSpec((_BN, D), row),
          pl.BlockSpec((_BN, D), row),
          pl.BlockSpec((_BN, D + G), row),
          pl.BlockSpec((_BN, D + G), row),
      ],
      out_shape=[
          jax.ShapeDtypeStruct((N, D), jnp.float32),
          jax.ShapeDtypeStruct((N, D), jnp.float32),
          jax.ShapeDtypeStruct((N, D + G), jnp.float32),
          jax.ShapeDtypeStruct((N, D + G), jnp.float32),
      ],
  )(t2, s2, x16, emb32, wt1, bt1, wt2, bt2, w1a, w1b)


@functools.cache
def _sc_mesh():
  return plsc.VectorSubcoreMesh(
      core_axis_name="c", subcore_axis_name="s", num_cores=NC, num_subcores=NS)


@functools.cache
def _make_sc_gather():
  @functools.partial(
      pl.kernel,
      out_type=[
          jax.ShapeDtypeStruct((E, D), jnp.float32),
          jax.ShapeDtypeStruct((E, G), jnp.float32),
      ],
      mesh=_sc_mesh(),
      compiler_params=pltpu.CompilerParams(use_tc_tiling_on_sc=False),
      scratch_types=[
          pltpu.VMEM((NCHUNK, CH), jnp.int32),
          pltpu.VMEM((NCHUNK, CH), jnp.int32),
          pltpu.VMEM((CH, D + G), jnp.float32),
          pltpu.VMEM((CH, D + G), jnp.float32),
          pltpu.VMEM((CH, D + G), jnp.float32),
          pltpu.VMEM((CH, D + G), jnp.float32),
          pltpu.SemaphoreType.DMA,
          pltpu.SemaphoreType.DMA,
      ],
  )
  def _sc_gather_k(t1_hbm, t2_hbm, src3_hbm, dst3_hbm,
                   ms_out, cd_out,
                   si_all, di_all, a0, b0, a1, b1,
                   sem0, sem1):
    wid = lax.axis_index("s") * NC + lax.axis_index("c")
    base = wid * EPW
    pltpu.sync_copy(src3_hbm.at[wid], si_all)
    pltpu.sync_copy(dst3_hbm.at[wid], di_all)

    def start(ch, a_v, b_v, sem):
      pltpu.async_copy(t1_hbm.at[si_all.at[ch]], a_v, sem)
      pltpu.async_copy(t2_hbm.at[di_all.at[ch]], b_v, sem)

    def wait(a_v, b_v, sem):
      pltpu.make_async_copy(t1_hbm.at[si_all.at[0]], a_v, sem).wait()
      pltpu.make_async_copy(t2_hbm.at[di_all.at[0]], b_v, sem).wait()

    def add_store(ch, a_v, b_v):
      def add_row(r5, carry):
        for rr in range(5):
          r = r5 * 5 + rr
          for k in range((D + G) // 16):
            sl = pl.ds(k * 16, 16)
            a_v[r, sl] = a_v[r, sl] + b_v[r, sl]
        return carry

      lax.fori_loop(0, CH // 5, add_row, None)
      rows = pl.ds(base + ch * CH, CH)
      pltpu.sync_copy(a_v.at[:, pl.ds(0, D)], ms_out.at[rows])
      pltpu.sync_copy(a_v.at[:, pl.ds(D, G)], cd_out.at[rows])

    start(0, a0, b0, sem0)

    def pair(j, _):
      c0 = 2 * j
      start(c0 + 1, a1, b1, sem1)
      wait(a0, b0, sem0)
      add_store(c0, a0, b0)

      @pl.when(j < HALF - 1)
      def _prefetch():
        start(c0 + 2, a0, b0, sem0)

      wait(a1, b1, sem1)
      add_store(c0 + 1, a1, b1)
      return _

    lax.fori_loop(0, HALF, pair, None)

  return _sc_gather_k


def _sc_gather(t1, t2_tab, src3, dst3):
  return _make_sc_gather()(t1, t2_tab, src3, dst3)


def _s3_body(ms_ref, cd_ref, a16, b16, wr48, br, w1c, be1, we2, be2, wc,
             m_o, cm_o):
  z = cd_ref[...]
  prod = _dot(z, a16[...]) * _dot(z, b16[...])
  rfeat = _silu(_dot(prod, wr48[...]) + br[...])
  u = ms_ref[...] + _dot(rfeat, w1c[...]) + be1[...]
  m = _silu(_dot(_silu(u), we2[...]) + be2[...])
  wgt = jnp.sum(m * wc[...], axis=1, keepdims=True)
  cm = z * wgt
  lane = lax.broadcasted_iota(jnp.int32, (_BE, G), 1)
  m_o[...] = m
  cm_o[...] = jnp.where(lane == 12, 1.0, cm)


def _edge_mlp(ms, cd, a16, b16, wr48, br, w1c, be1, we2, be2, wc):
  nb = E // _BE
  row = lambda i: (i, 0)
  rep = lambda i: (0, 0)
  return pl.pallas_call(
      _s3_body,
      grid=(nb,),
      in_specs=[
          pl.BlockSpec((_BE, D), row),
          pl.BlockSpec((_BE, G), row),
          pl.BlockSpec((G, D), rep),
          pl.BlockSpec((G, D), rep),
          pl.BlockSpec((D, D), rep),
          pl.BlockSpec((1, D), rep),
          pl.BlockSpec((D, D), rep),
          pl.BlockSpec((1, D), rep),
          pl.BlockSpec((D, D), rep),
          pl.BlockSpec((1, D), rep),
          pl.BlockSpec((1, D), rep),
      ],
      out_specs=[
          pl.BlockSpec((_BE, D), row),
          pl.BlockSpec((_BE, G), row),
      ],
      out_shape=[
          jax.ShapeDtypeStruct((E, D), jnp.float32),
          jax.ShapeDtypeStruct((E, G), jnp.float32),
      ],
  )(ms, cd, a16, b16, wr48, br, w1c, be1, we2, be2, wc)


@functools.cache
def _make_sc_scatter():
  @functools.partial(
      pl.kernel,
      out_type=[
          jax.ShapeDtypeStruct((NC, N, D), jnp.float32),
          jax.ShapeDtypeStruct((NC, N, G), jnp.float32),
      ],
      mesh=_sc_mesh(),
      compiler_params=pltpu.CompilerParams(use_tc_tiling_on_sc=False),
      scratch_types=[
          pltpu.VMEM((NCHS, CHS), jnp.int32),
          pltpu.VMEM((CHS, D), jnp.float32),
          pltpu.VMEM((CHS, D), jnp.float32),
          pltpu.VMEM((CHS, G), jnp.float32),
          pltpu.VMEM((CHS, G), jnp.float32),
          pltpu.VMEM_SHARED((N, D), jnp.float32),
          pltpu.VMEM_SHARED((N, G), jnp.float32),
          pltpu.SemaphoreType.DMA,
          pltpu.SemaphoreType.DMA,
      ],
  )
  def _sc_scatter_k(m_hbm, cm_hbm, dst3_hbm, zd_hbm, zg_hbm,
                    outd_hbm, outg_hbm,
                    idx_all, m0, m1, c0_v, c1_v, shd, shg, sem0, sem1):
    c = lax.axis_index("c")
    s = lax.axis_index("s")
    wid = s * NC + c
    rows0 = s * RPS
    pltpu.sync_copy(zd_hbm.at[pl.ds(rows0, RPS)], shd.at[pl.ds(rows0, RPS)])
    pltpu.sync_copy(zg_hbm.at[pl.ds(rows0, RPS)], shg.at[pl.ds(rows0, RPS)])
    pltpu.sync_copy(dst3_hbm.at[wid], idx_all)
    plsc.subcore_barrier()

    base = wid * EPW

    def start(ch, m_v, c_v, sem):
      pltpu.async_copy(m_hbm.at[pl.ds(base + ch * CHS, CHS)], m_v, sem)
      pltpu.async_copy(cm_hbm.at[pl.ds(base + ch * CHS, CHS)], c_v, sem)

    def wait(m_v, c_v, sem):
      pltpu.make_async_copy(m_hbm.at[pl.ds(0, CHS)], m_v, sem).wait()
      pltpu.make_async_copy(cm_hbm.at[pl.ds(0, CHS)], c_v, sem).wait()

    start(0, m0, c0_v, sem0)

    def pair(j, _):
      c0 = 2 * j
      start(c0 + 1, m1, c1_v, sem1)
      wait(m0, c0_v, sem0)
      pltpu.sync_copy(m0, shd.at[idx_all.at[c0]], add=True)
      pltpu.sync_copy(c0_v, shg.at[idx_all.at[c0]], add=True)

      @pl.when(j < HALFS - 1)
      def _prefetch():
        start(c0 + 2, m0, c0_v, sem0)

      wait(m1, c1_v, sem1)
      pltpu.sync_copy(m1, shd.at[idx_all.at[c0 + 1]], add=True)
      pltpu.sync_copy(c1_v, shg.at[idx_all.at[c0 + 1]], add=True)
      return _

    lax.fori_loop(0, HALFS, pair, None)
    plsc.subcore_barrier()
    pltpu.sync_copy(shd.at[pl.ds(rows0, RPS)], outd_hbm.at[c, pl.ds(rows0, RPS)])
    pltpu.sync_copy(shg.at[pl.ds(rows0, RPS)], outg_hbm.at[c, pl.ds(rows0, RPS)])

  return _sc_scatter_k


def _sc_scatter(m, cm, dst3s, zd, zg):
  return _make_sc_scatter()(m, cm, dst3s, zd, zg)


def _s5_body(hn_ref, h0_ref, x_ref, pd_ref, pg_ref, wn1a, wn1b, bn1, wn2, bn2,
             wp, bp, wf1, bf1, wf2, logits_o, xout_o):
  hn = hn_ref[...]
  h0 = h0_ref[...]
  pd = pd_ref[...]
  pg = pg_ref[...]
  agg = pd[0] + pd[1]
  cm = pg[0] + pg[1]
  hmid = _silu(_dot(hn, wn1a[...]) + _dot(agg, wn1b[...]) + bn1[...])
  hout = hn + _dot(hmid, wn2[...]) + bn2[...]
  cnt = jnp.maximum(cm[:, 12:13], 1.0)
  lane = lax.broadcasted_iota(jnp.int32, (_BN, G), 1)
  xout_o[...] = x_ref[...] + jnp.where(lane < 12, cm / cnt, 0.0)
  proj = _dot(_silu(hout), wp[...]) + bp[...]
  gate = h0 * jax.nn.sigmoid(proj)
  l1 = _silu(_dot(_silu(gate), wf1[...]) + bf1[...])
  logits_o[...] = _dot(l1, wf2[...])


def _node_post(hn, h0, x16, pd, pg, wn1a, wn1b, bn1, wn2, bn2,
               wp, bp, wf1, bf1, wf2):
  nb = N // _BN
  row = lambda i: (i, 0)
  rep = lambda i: (0, 0)
  return pl.pallas_call(
      _s5_body,
      grid=(nb,),
      in_specs=[
          pl.BlockSpec((_BN, D), row),
          pl.BlockSpec((_BN, D), row),
          pl.BlockSpec((_BN, G), row),
          pl.BlockSpec((NC, _BN, D), lambda i: (0, i, 0)),
          pl.BlockSpec((NC, _BN, G), lambda i: (0, i, 0)),
          pl.BlockSpec((D, D), rep),
          pl.BlockSpec((D, D), rep),
          pl.BlockSpec((1, D), rep),
          pl.BlockSpec((D, D), rep),
          pl.BlockSpec((1, D), rep),
          pl.BlockSpec((D, D), rep),
          pl.BlockSpec((1, D), rep),
          pl.BlockSpec((D, D), rep),
          pl.BlockSpec((1, D), rep),
          pl.BlockSpec((D, D), rep),
      ],
      out_specs=[
          pl.BlockSpec((_BN, D), row),
          pl.BlockSpec((_BN, G), row),
      ],
      out_shape=[
          jax.ShapeDtypeStruct((N, D), jnp.float32),
          jax.ShapeDtypeStruct((N, G), jnp.float32),
      ],
  )(hn, h0, x16, pd, pg, wn1a, wn1b, bn1, wn2, bn2, wp, bp, wf1, bf1, wf2)


def _radial_selectors():
  a = np.zeros((G, D), np.float32)
  b = np.zeros((G, D), np.float32)
  for c in range(4):
    for dd in range(4):
      for i in range(3):
        p = (c * 4 + dd) * 3 + i
        a[3 * c + i, p] = 1.0
        b[3 * dd + i, p] = 1.0
  return a, b


_A16_NP, _B16_NP = _radial_selectors()
_CD_IDX = np.repeat(np.arange(16), 3)


def kernel(X, t, S, edge_index, params):
  p = params
  x16 = jnp.pad(X.reshape(N, 12), ((0, 0), (0, 4)))
  t2 = t.reshape(N, 1).astype(jnp.float32)
  s2 = S.reshape(N, 1).astype(jnp.int32)
  src = edge_index[0].astype(jnp.int32)
  dst = edge_index[1].astype(jnp.int32)
  src3 = src.reshape(NW, NCHUNK, CH)
  dst3 = dst.reshape(NW, NCHUNK, CH)
  dst3s = dst.reshape(NW, NCHS, CHS)

  emb32 = jnp.pad(p["embed"], ((0, 32 - (NUM_CLASSES + 1)), (0, 0)))
  w1a = p["W_e1"][:D]
  w1b = p["W_e1"][D:2 * D]
  w1c = p["W_e1"][2 * D:]
  wr48 = jnp.pad(p["W_r"][_CD_IDX], ((0, D - 48), (0, 0)))
  row128 = lambda v: v.reshape(1, D)

  hn, h0, t1, t2_tab = _node_pre(
      t2, s2, x16, emb32,
      p["W_t1"], row128(p["b_t1"]), p["W_t2"], row128(p["b_t2"]), w1a, w1b)

  ms, cd = _sc_gather(t1, t2_tab, src3, dst3)

  m, cm = _edge_mlp(
      ms, cd, jnp.asarray(_A16_NP), jnp.asarray(_B16_NP), wr48,
      row128(p["b_r"]), w1c, row128(p["b_e1"]),
      p["W_e2"], row128(p["b_e2"]), p["W_c"].reshape(1, D))

  pd, pg = _sc_scatter(m, cm, dst3s,
                       jnp.zeros((N, D), jnp.float32),
                       jnp.zeros((N, G), jnp.float32))

  wf2 = jnp.pad(p["W_f2"], ((0, 0), (0, D - NUM_CLASSES)))
  logits128, xout16 = _node_post(
      hn, h0, x16, pd, pg,
      p["W_n1"][:D], p["W_n1"][D:], row128(p["b_n1"]),
      p["W_n2"], row128(p["b_n2"]),
      p["W_p"], row128(p["b_p"]),
      p["W_f1"], row128(p["b_f1"]), wf2)

  logits = logits128[:, :NUM_CLASSES] + p["b_f2"]
  xout = xout16[:, :12].reshape(N, C, 3)
  return logits, xout
